# baseline (speedup 1.0000x reference)
import functools

import jax
import jax.numpy as jnp
from jax import lax
from jax.experimental import pallas as pl
from jax.experimental.pallas import tpu as pltpu
from jax.experimental.pallas import tpu_sc as plsc

HIDDEN = 1024
B = 4 * 4096
NC = 2
NS = 16
NW = NC * NS
RPW = B // NW
CH = 64
PPC = CH // 2
NCHUNK = RPW // CH
L = 16
NT = HIDDEN // L

_mesh = plsc.VectorSubcoreMesh(core_axis_name="c", subcore_axis_name="s")


@functools.partial(
    pl.kernel,
    out_type=jax.ShapeDtypeStruct((B, HIDDEN), jnp.float16),
    mesh=_mesh,
    compiler_params=pltpu.CompilerParams(use_tc_tiling_on_sc=True),
    scratch_types=[
        pltpu.VMEM((RPW, L), jnp.float16),
        pltpu.VMEM((2, HIDDEN), jnp.float16),
        pltpu.VMEM((2, HIDDEN), jnp.float16),
        pltpu.VMEM((CH, HIDDEN), jnp.float16),
        pltpu.VMEM((CH, HIDDEN), jnp.float16),
        pltpu.SemaphoreType.DMA,
        pltpu.SemaphoreType.DMA,
    ],
)
def _lookup(m_hbm, t0_hbm, t1_hbm, out_hbm, m_v, t0_v, t1_v, buf0, buf1,
            s0, s1):
    wid = lax.axis_index("s") * NC + lax.axis_index("c")
    rbase = wid * RPW
    pltpu.sync_copy(m_hbm.at[pl.ds(rbase, RPW)], m_v)
    pltpu.sync_copy(t0_hbm, t0_v)
    pltpu.sync_copy(t1_hbm, t1_v)

    bufs = (buf0, buf1)
    ssems = (s0, s1)

    UNROLL = 4

    def build(buf, c0):
        def group(jg, carry):
            for ju in range(UNROLL):
                jp = jg * UNROLL + ju
                r = 2 * (c0 * PPC + jp)
                pred = (m_v[pl.ds(r, 2), :]
                        != jnp.zeros((2, L), jnp.float16))
                for t in range(NT):
                    sl = pl.ds(t * L, L)
                    buf[pl.ds(2 * jp, 2), sl] = jnp.where(
                        pred, t1_v[:, sl], t0_v[:, sl])
            return carry
        lax.fori_loop(0, PPC // UNROLL, group, 0)

    def wait_out(b):
        pltpu.make_async_copy(
            bufs[b], out_hbm.at[pl.ds(0, CH)], ssems[b]).wait()

    for b in range(2):
        def body(cp, carry, b=b):
            c0 = 2 * cp + b

            @pl.when(cp >= 1)
            def _():
                wait_out(b)

            build(bufs[b], c0)
            pltpu.async_copy(
                bufs[b], out_hbm.at[pl.ds(rbase + c0 * CH, CH)], ssems[b])
            return carry

        lax.fori_loop(0, NCHUNK // 2, body, 0)
    wait_out(0)
    wait_out(1)


def kernel(token_type_ids, token_type_table):
    ids = jnp.reshape(token_type_ids, (B,)).astype(jnp.int32)
    m = jnp.broadcast_to(ids.astype(jnp.float16)[:, None], (B, L))
    t0 = jnp.broadcast_to(token_type_table[0], (2, HIDDEN))
    t1 = jnp.broadcast_to(token_type_table[1], (2, HIDDEN))
    return _lookup(m, t0, t1)


# speedup vs baseline: 1.0091x; 1.0091x over previous
import functools

import jax
import jax.numpy as jnp
from jax import lax
from jax.experimental import pallas as pl
from jax.experimental.pallas import tpu as pltpu
from jax.experimental.pallas import tpu_sc as plsc

HIDDEN = 1024
B = 4 * 4096
NC = 2
NS = 16
NW = NC * NS
RPW = B // NW
CH = 64
PPC = CH // 2
NCHUNK = RPW // CH
L = 16
NT = HIDDEN // L

_mesh = plsc.VectorSubcoreMesh(core_axis_name="c", subcore_axis_name="s")


@functools.partial(
    pl.kernel,
    out_type=jax.ShapeDtypeStruct((B, HIDDEN), jnp.float16),
    mesh=_mesh,
    compiler_params=pltpu.CompilerParams(use_tc_tiling_on_sc=True),
    scratch_types=[
        pltpu.VMEM((RPW, L), jnp.float16),
        pltpu.VMEM((2, HIDDEN), jnp.float16),
        pltpu.VMEM((2, HIDDEN), jnp.float16),
        pltpu.VMEM((CH, HIDDEN), jnp.float16),
        pltpu.VMEM((CH, HIDDEN), jnp.float16),
        pltpu.SemaphoreType.DMA,
        pltpu.SemaphoreType.DMA,
    ],
)
def _lookup(m_hbm, t0_hbm, t1_hbm, out_hbm, m_v, t0_v, t1_v, buf0, buf1,
            s0, s1):
    wid = lax.axis_index("s") * NC + lax.axis_index("c")
    rbase = wid * RPW
    pltpu.sync_copy(m_hbm.at[pl.ds(rbase, RPW)], m_v)
    pltpu.sync_copy(t0_hbm, t0_v)
    pltpu.sync_copy(t1_hbm, t1_v)

    bufs = (buf0, buf1)
    ssems = (s0, s1)

    def build(buf, c0):
        def pair(jp, carry):
            r = 2 * (c0 * PPC + jp)
            pred = m_v[pl.ds(r, 2), :] != jnp.zeros((2, L), jnp.float16)
            for t in range(NT):
                sl = pl.ds(t * L, L)
                buf[pl.ds(2 * jp, 2), sl] = jnp.where(
                    pred, t1_v[:, sl], t0_v[:, sl])
            return carry
        lax.fori_loop(0, PPC, pair, 0)

    def wait_out(b):
        pltpu.make_async_copy(
            bufs[b], out_hbm.at[pl.ds(0, CH)], ssems[b]).wait()

    for b in range(2):
        def body(cp, carry, b=b):
            c0 = 2 * cp + b

            @pl.when(cp >= 1)
            def _():
                wait_out(b)

            build(bufs[b], c0)
            pltpu.async_copy(
                bufs[b], out_hbm.at[pl.ds(rbase + c0 * CH, CH)], ssems[b])
            return carry

        lax.fori_loop(0, NCHUNK // 2, body, 0)
    wait_out(0)
    wait_out(1)


def kernel(token_type_ids, token_type_table):
    ids = jnp.reshape(token_type_ids, (B,)).astype(jnp.int32)
    m = jnp.broadcast_to(ids.astype(jnp.float16)[:, None], (B, L))
    t0 = jnp.broadcast_to(token_type_table[0], (2, HIDDEN))
    t1 = jnp.broadcast_to(token_type_table[1], (2, HIDDEN))
    return _lookup(m, t0, t1)
